# baseline (speedup 1.0000x reference)
import jax
import jax.numpy as jnp
from jax.experimental import pallas as pl
from jax.experimental.pallas import tpu as pltpu

_R = 8
_C = 2048


def _body(bins_ref, x_ref, o_ref):
    x = x_ref[...]
    acc = jnp.zeros(x.shape, jnp.int32)
    for k in range(1, 9):
        acc = acc + (x > bins_ref[k]).astype(jnp.int32)
    ch = jax.lax.broadcasted_iota(jnp.int32, (x.shape[0], x.shape[1], 10), 2)
    hit = (acc[:, :, None] + 1 == ch) | (ch == 0)
    o_ref[...] = jnp.where(hit, 1.0, 0.0).astype(jnp.float32)


def kernel(x, bins):
    M, N = x.shape
    D = bins.shape[0]
    grid = (M // _R, N // _C)
    return pl.pallas_call(
        _body,
        grid=grid,
        in_specs=[
            pl.BlockSpec(memory_space=pltpu.SMEM),
            pl.BlockSpec((_R, _C), lambda i, j: (i, j)),
        ],
        out_specs=pl.BlockSpec((_R, _C, D), lambda i, j: (i, j, 0)),
        out_shape=jax.ShapeDtypeStruct((M, N, D), jnp.float32),
        compiler_params=pltpu.CompilerParams(
            dimension_semantics=("parallel", "parallel"),
        ),
    )(bins, x)


# speedup vs baseline: 30.9134x; 30.9134x over previous
import jax
import jax.numpy as jnp
from jax.experimental import pallas as pl
from jax.experimental.pallas import tpu as pltpu

_R = 32
_D = 10


def _body(bins_ref, x_ref, o_ref):
    x = x_ref[...]
    one = jnp.ones(x.shape, jnp.float32)
    zero = jnp.zeros(x.shape, jnp.float32)
    o_ref[0, :, :] = one
    above = [x > bins_ref[k] for k in range(1, 9)]
    o_ref[1, :, :] = jnp.where(above[0], zero, one)
    for c in range(2, 9):
        o_ref[c, :, :] = jnp.where(above[c - 2] & (~above[c - 1]), one, zero)
    o_ref[9, :, :] = jnp.where(above[7], one, zero)


def kernel(x, bins):
    M, N = x.shape
    grid = (M // _R,)
    out = pl.pallas_call(
        _body,
        grid=grid,
        in_specs=[
            pl.BlockSpec(memory_space=pltpu.SMEM),
            pl.BlockSpec((_R, N), lambda i: (i, 0)),
        ],
        out_specs=pl.BlockSpec((_D, _R, N), lambda i: (0, i, 0)),
        out_shape=jax.ShapeDtypeStruct((_D, M, N), jnp.float32),
        compiler_params=pltpu.CompilerParams(
            dimension_semantics=("parallel",),
        ),
    )(bins, x)
    return jnp.transpose(out, (1, 2, 0))
